# baseline (speedup 1.0000x reference)
import functools

import jax
import jax.numpy as jnp
from jax import lax
from jax.experimental import pallas as pl
from jax.experimental.pallas import tpu as pltpu
from jax.experimental.pallas import tpu_sc as plsc

E = 160000
D = 256
DW = D // 2
HW = DW // 2
NC = 2
NS = 16
NW = NC * NS
EPW = E // NW
B = 192
NBW = EPW // B
TAIL = EPW - NBW * B
TPAD = 16
EBUF = EPW + (TPAD - TAIL)


def _body(lig_hbm, prot_hbm, src_hbm, dst_hbm, out_hbm,
          idxs, idxd, ligA, protA, ligB, protB, outb,
          sla, spa, slb, spb):
    w = lax.axis_index("s") * NC + lax.axis_index("c")
    ebase = w * EPW
    lanes = lax.iota(jnp.int32, 16)

    pltpu.sync_copy(src_hbm.at[pl.ds(ebase, EPW)], idxs.at[pl.ds(0, EPW)])
    pltpu.sync_copy(dst_hbm.at[pl.ds(ebase, EPW)], idxd.at[pl.ds(0, EPW)])
    vs = idxs[pl.ds(EBUF - 16, 16)]
    idxs[pl.ds(EBUF - 16, 16)] = jnp.where(lanes < TAIL, vs, 0)
    vd = idxd[pl.ds(EBUF - 16, 16)]
    idxd[pl.ds(EBUF - 16, 16)] = jnp.where(lanes < TAIL, vd, 0)

    def issue(off, n, ligbuf, protbuf, sem_l, sem_p):
        cl = pltpu.async_copy(lig_hbm.at[idxs.at[pl.ds(off, n)]],
                              ligbuf, sem_l)
        cp = pltpu.async_copy(prot_hbm.at[idxd.at[pl.ds(off, n)]],
                              protbuf, sem_p)
        return cl, cp

    def wait(off, n, ligbuf, protbuf, sem_l, sem_p):
        pltpu.make_async_copy(lig_hbm.at[idxs.at[pl.ds(off, n)]],
                              ligbuf, sem_l).wait()
        pltpu.make_async_copy(prot_hbm.at[idxd.at[pl.ds(off, n)]],
                              protbuf, sem_p).wait()

    def compute_block(obase, ligbuf, protbuf, ngroups):
        def group_body(g, carry):
            def edge_pair(t, carry2):
                var_acc, ss_acc = carry2
                for c in range(2):
                    m = t * 2 + c
                    e = g * 16 + m
                    accv = jnp.zeros((16,), jnp.float32)
                    for j in range(HW // 16):
                        l = ligbuf[e, pl.ds(j * 16, 16)]
                        p = protbuf[e, pl.ds(j * 16, 16)]
                        lb = plsc.bitcast(l, jnp.bfloat16)
                        pb = plsc.bitcast(p, jnp.bfloat16)
                        ad = jnp.abs(lb - pb)
                        a0, a1 = plsc.unpack(ad, format=plsc.PackFormat.INTERLEAVED)
                        accv = accv + a0 + a1
                    var = jnp.sum(accv)
                    accs = jnp.zeros((16,), jnp.float32)
                    for j in range(HW // 16, DW // 16):
                        l = ligbuf[e, pl.ds(j * 16, 16)]
                        p = protbuf[e, pl.ds(j * 16, 16)]
                        lb = plsc.bitcast(l, jnp.bfloat16)
                        pb = plsc.bitcast(p, jnp.bfloat16)
                        d0, d1 = plsc.unpack(lb - pb, format=plsc.PackFormat.INTERLEAVED)
                        accs = accs + d0 * d0
                        accs = accs + d1 * d1
                    ss = jnp.sum(accs)
                    msk = lanes == m
                    var_acc = jnp.where(msk, var, var_acc)
                    ss_acc = jnp.where(msk, ss, ss_acc)
                return var_acc, ss_acc

            z16 = jnp.zeros((16,), jnp.float32)
            var_acc, ss_acc = lax.fori_loop(0, 8, edge_pair, (z16, z16))

            i32 = plsc.bitcast(ss_acc, jnp.int32)
            i32 = jnp.int32(0x5F3759DF) - lax.shift_right_arithmetic(i32, 1)
            y = plsc.bitcast(i32, jnp.float32)
            for _ in range(3):
                y = y * (1.5 - 0.5 * ss_acc * y * y)
            mu_vec = jnp.where(ss_acc > 0.0, ss_acc * y, 0.0)

            iv = (obase + g * 16 + lanes) * 2
            plsc.store_scatter(outb, [iv], mu_vec)
            plsc.store_scatter(outb, [iv + 1], var_acc)
            return carry

        lax.fori_loop(0, ngroups, group_body, 0)

    issue(0, B, ligA, protA, sla, spa)

    def pair_body(p, carry):
        k = 2 * p
        issue((k + 1) * B, B, ligB, protB, slb, spb)
        wait(k * B, B, ligA, protA, sla, spa)
        compute_block(k * B, ligA, protA, B // 16)

        @pl.when(k + 2 < NBW)
        def _():
            issue((k + 2) * B, B, ligA, protA, sla, spa)

        wait((k + 1) * B, B, ligB, protB, slb, spb)
        compute_block((k + 1) * B, ligB, protB, B // 16)
        return carry

    lax.fori_loop(0, NBW // 2, pair_body, 0)

    tl = ligA.at[pl.ds(0, TPAD), :]
    tp = protA.at[pl.ds(0, TPAD), :]
    issue(NBW * B, TPAD, tl, tp, sla, spa)
    wait(NBW * B, TPAD, tl, tp, sla, spa)
    compute_block(NBW * B, ligA, protA, 1)

    pltpu.sync_copy(outb.at[pl.ds(0, 2 * EPW)],
                    out_hbm.at[pl.ds(2 * ebase, 2 * EPW)])


@functools.partial(
    pl.kernel,
    out_type=jax.ShapeDtypeStruct((2 * E,), jnp.float32),
    mesh=plsc.VectorSubcoreMesh(core_axis_name="c", subcore_axis_name="s"),
    compiler_params=pltpu.CompilerParams(needs_layout_passes=False),
    scratch_types=[
        pltpu.VMEM((EBUF,), jnp.int32),
        pltpu.VMEM((EBUF,), jnp.int32),
        pltpu.VMEM((B, DW), jnp.float32),
        pltpu.VMEM((B, DW), jnp.float32),
        pltpu.VMEM((B, DW), jnp.float32),
        pltpu.VMEM((B, DW), jnp.float32),
        pltpu.VMEM((2 * EBUF,), jnp.float32),
        pltpu.SemaphoreType.DMA,
        pltpu.SemaphoreType.DMA,
        pltpu.SemaphoreType.DMA,
        pltpu.SemaphoreType.DMA,
    ],
)
def _sc_kernel(lig, prot, src, dst, out,
               idxs, idxd, ligA, protA, ligB, protB, outb,
               sla, spa, slb, spb):
    _body(lig, prot, src, dst, out,
          idxs, idxd, ligA, protA, ligB, protB, outb,
          sla, spa, slb, spb)


def _pack_words(x):
    xb = lax.bitcast_convert_type(x.astype(jnp.bfloat16), jnp.uint16)
    x32 = xb.astype(jnp.uint32)
    h = x.shape[1] // 4
    w_var = x32[:, 0:h] | (x32[:, h:2 * h] << 16)
    w_ss = x32[:, 2 * h:3 * h] | (x32[:, 3 * h:4 * h] << 16)
    return lax.bitcast_convert_type(
        jnp.concatenate([w_var, w_ss], axis=1), jnp.float32)


def kernel(ligand_h, protein_h, edge_index):
    ei = edge_index.astype(jnp.int32)
    out = _sc_kernel(_pack_words(ligand_h), _pack_words(protein_h),
                     ei[0], ei[1])
    return out.reshape(E, 2)


# speedup vs baseline: 1.8377x; 1.8377x over previous
import functools

import jax
import jax.numpy as jnp
from jax import lax
from jax.experimental import pallas as pl
from jax.experimental.pallas import tpu as pltpu
from jax.experimental.pallas import tpu_sc as plsc

E = 160000
D = 256
DW = D // 2
HW = DW // 2
NC = 2
NS = 16
NW = NC * NS
EPW = E // NW
B = 192
NBW = EPW // B
TAIL = EPW - NBW * B
TPAD = 16
EBUF = EPW + (TPAD - TAIL)


def _body(lig_hbm, prot_hbm, src_hbm, dst_hbm, mu_hbm, var_hbm,
          idxs, idxd, ligA, protA, ligB, protB, mub, varb,
          sla, spa, slb, spb):
    w = lax.axis_index("s") * NC + lax.axis_index("c")
    ebase = w * EPW
    lanes = lax.iota(jnp.int32, 16)

    pltpu.sync_copy(src_hbm.at[pl.ds(ebase, EPW)], idxs.at[pl.ds(0, EPW)])
    pltpu.sync_copy(dst_hbm.at[pl.ds(ebase, EPW)], idxd.at[pl.ds(0, EPW)])
    vs = idxs[pl.ds(EBUF - 16, 16)]
    idxs[pl.ds(EBUF - 16, 16)] = jnp.where(lanes < TAIL, vs, 0)
    vd = idxd[pl.ds(EBUF - 16, 16)]
    idxd[pl.ds(EBUF - 16, 16)] = jnp.where(lanes < TAIL, vd, 0)

    def issue(off, n, ligbuf, protbuf, sem_l, sem_p):
        cl = pltpu.async_copy(lig_hbm.at[idxs.at[pl.ds(off, n)]],
                              ligbuf, sem_l)
        cp = pltpu.async_copy(prot_hbm.at[idxd.at[pl.ds(off, n)]],
                              protbuf, sem_p)
        return cl, cp

    def wait(off, n, ligbuf, protbuf, sem_l, sem_p):
        pltpu.make_async_copy(lig_hbm.at[idxs.at[pl.ds(off, n)]],
                              ligbuf, sem_l).wait()
        pltpu.make_async_copy(prot_hbm.at[idxd.at[pl.ds(off, n)]],
                              protbuf, sem_p).wait()

    def compute_block(obase, ligbuf, protbuf, ngroups):
        def group_body(g, carry):
            def edge_pair(t, carry2):
                var_acc, ss_acc = carry2
                for c in range(2):
                    m = t * 2 + c
                    e = g * 16 + m
                    accv = jnp.zeros((16,), jnp.float32)
                    for j in range(HW // 16):
                        l = ligbuf[e, pl.ds(j * 16, 16)]
                        p = protbuf[e, pl.ds(j * 16, 16)]
                        lb = plsc.bitcast(l, jnp.bfloat16)
                        pb = plsc.bitcast(p, jnp.bfloat16)
                        ad = jnp.abs(lb - pb)
                        a0, a1 = plsc.unpack(ad, format=plsc.PackFormat.INTERLEAVED)
                        accv = accv + a0 + a1
                    var = jnp.sum(accv)
                    accs = jnp.zeros((16,), jnp.float32)
                    for j in range(HW // 16, DW // 16):
                        l = ligbuf[e, pl.ds(j * 16, 16)]
                        p = protbuf[e, pl.ds(j * 16, 16)]
                        lb = plsc.bitcast(l, jnp.bfloat16)
                        pb = plsc.bitcast(p, jnp.bfloat16)
                        d0, d1 = plsc.unpack(lb - pb, format=plsc.PackFormat.INTERLEAVED)
                        accs = accs + d0 * d0
                        accs = accs + d1 * d1
                    ss = jnp.sum(accs)
                    msk = lanes == m
                    var_acc = jnp.where(msk, var, var_acc)
                    ss_acc = jnp.where(msk, ss, ss_acc)
                return var_acc, ss_acc

            z16 = jnp.zeros((16,), jnp.float32)
            var_acc, ss_acc = lax.fori_loop(0, 8, edge_pair, (z16, z16))

            i32 = plsc.bitcast(ss_acc, jnp.int32)
            i32 = jnp.int32(0x5F3759DF) - lax.shift_right_arithmetic(i32, 1)
            y = plsc.bitcast(i32, jnp.float32)
            for _ in range(3):
                y = y * (1.5 - 0.5 * ss_acc * y * y)
            mu_vec = jnp.where(ss_acc > 0.0, ss_acc * y, 0.0)

            mub[pl.ds(obase + g * 16, 16)] = mu_vec
            varb[pl.ds(obase + g * 16, 16)] = var_acc
            return carry

        lax.fori_loop(0, ngroups, group_body, 0)

    issue(0, B, ligA, protA, sla, spa)

    def pair_body(p, carry):
        k = 2 * p
        issue((k + 1) * B, B, ligB, protB, slb, spb)
        wait(k * B, B, ligA, protA, sla, spa)
        compute_block(k * B, ligA, protA, B // 16)

        @pl.when(k + 2 < NBW)
        def _():
            issue((k + 2) * B, B, ligA, protA, sla, spa)

        wait((k + 1) * B, B, ligB, protB, slb, spb)
        compute_block((k + 1) * B, ligB, protB, B // 16)
        return carry

    lax.fori_loop(0, NBW // 2, pair_body, 0)

    tl = ligA.at[pl.ds(0, TPAD), :]
    tp = protA.at[pl.ds(0, TPAD), :]
    issue(NBW * B, TPAD, tl, tp, sla, spa)
    wait(NBW * B, TPAD, tl, tp, sla, spa)
    compute_block(NBW * B, ligA, protA, 1)

    pltpu.sync_copy(mub.at[pl.ds(0, EPW)], mu_hbm.at[pl.ds(ebase, EPW)])
    pltpu.sync_copy(varb.at[pl.ds(0, EPW)], var_hbm.at[pl.ds(ebase, EPW)])


@functools.partial(
    pl.kernel,
    out_type=(
        jax.ShapeDtypeStruct((E,), jnp.float32),
        jax.ShapeDtypeStruct((E,), jnp.float32),
    ),
    mesh=plsc.VectorSubcoreMesh(core_axis_name="c", subcore_axis_name="s"),
    compiler_params=pltpu.CompilerParams(needs_layout_passes=False),
    scratch_types=[
        pltpu.VMEM((EBUF,), jnp.int32),
        pltpu.VMEM((EBUF,), jnp.int32),
        pltpu.VMEM((B, DW), jnp.float32),
        pltpu.VMEM((B, DW), jnp.float32),
        pltpu.VMEM((B, DW), jnp.float32),
        pltpu.VMEM((B, DW), jnp.float32),
        pltpu.VMEM((EBUF,), jnp.float32),
        pltpu.VMEM((EBUF,), jnp.float32),
        pltpu.SemaphoreType.DMA,
        pltpu.SemaphoreType.DMA,
        pltpu.SemaphoreType.DMA,
        pltpu.SemaphoreType.DMA,
    ],
)
def _sc_kernel(lig, prot, src, dst, mu_out, var_out,
               idxs, idxd, ligA, protA, ligB, protB, mub, varb,
               sla, spa, slb, spb):
    _body(lig, prot, src, dst, mu_out, var_out,
          idxs, idxd, ligA, protA, ligB, protB, mub, varb,
          sla, spa, slb, spb)


def _pack_words(x):
    xb = lax.bitcast_convert_type(x.astype(jnp.bfloat16), jnp.uint16)
    x32 = xb.astype(jnp.uint32)
    h = x.shape[1] // 4
    w_var = x32[:, 0:h] | (x32[:, h:2 * h] << 16)
    w_ss = x32[:, 2 * h:3 * h] | (x32[:, 3 * h:4 * h] << 16)
    return lax.bitcast_convert_type(
        jnp.concatenate([w_var, w_ss], axis=1), jnp.float32)


def kernel(ligand_h, protein_h, edge_index):
    ei = edge_index.astype(jnp.int32)
    mu, var = _sc_kernel(_pack_words(ligand_h), _pack_words(protein_h),
                         ei[0], ei[1])
    return jnp.stack([mu, var], axis=0).T


# speedup vs baseline: 1.8447x; 1.0038x over previous
import functools

import jax
import jax.numpy as jnp
from jax import lax
from jax.experimental import pallas as pl
from jax.experimental.pallas import tpu as pltpu
from jax.experimental.pallas import tpu_sc as plsc

E = 160000
D = 256
DW = D // 2
HW = DW // 2
NC = 2
NS = 16
NW = NC * NS
EPW = E // NW
B = 208
NBW = EPW // B
TAIL = EPW - NBW * B
TPAD = 16
EBUF = EPW + (TPAD - TAIL)


def _body(lig_hbm, prot_hbm, src_hbm, dst_hbm, mu_hbm, var_hbm,
          idxs, idxd, ligA, protA, ligB, protB, mub, varb,
          sla, spa, slb, spb):
    w = lax.axis_index("s") * NC + lax.axis_index("c")
    ebase = w * EPW
    lanes = lax.iota(jnp.int32, 16)

    pltpu.sync_copy(src_hbm.at[pl.ds(ebase, EPW)], idxs.at[pl.ds(0, EPW)])
    pltpu.sync_copy(dst_hbm.at[pl.ds(ebase, EPW)], idxd.at[pl.ds(0, EPW)])
    vs = idxs[pl.ds(EBUF - 16, 16)]
    idxs[pl.ds(EBUF - 16, 16)] = jnp.where(lanes < TAIL, vs, 0)
    vd = idxd[pl.ds(EBUF - 16, 16)]
    idxd[pl.ds(EBUF - 16, 16)] = jnp.where(lanes < TAIL, vd, 0)

    def issue(off, n, ligbuf, protbuf, sem_l, sem_p):
        cl = pltpu.async_copy(lig_hbm.at[idxs.at[pl.ds(off, n)]],
                              ligbuf, sem_l)
        cp = pltpu.async_copy(prot_hbm.at[idxd.at[pl.ds(off, n)]],
                              protbuf, sem_p)
        return cl, cp

    def wait(off, n, ligbuf, protbuf, sem_l, sem_p):
        pltpu.make_async_copy(lig_hbm.at[idxs.at[pl.ds(off, n)]],
                              ligbuf, sem_l).wait()
        pltpu.make_async_copy(prot_hbm.at[idxd.at[pl.ds(off, n)]],
                              protbuf, sem_p).wait()

    def compute_block(obase, ligbuf, protbuf, ngroups):
        def group_body(g, carry):
            def edge_pair(t, carry2):
                var_acc, ss_acc = carry2
                for c in range(2):
                    m = t * 2 + c
                    e = g * 16 + m
                    accv = jnp.zeros((16,), jnp.float32)
                    for j in range(HW // 16):
                        l = ligbuf[e, pl.ds(j * 16, 16)]
                        p = protbuf[e, pl.ds(j * 16, 16)]
                        lb = plsc.bitcast(l, jnp.bfloat16)
                        pb = plsc.bitcast(p, jnp.bfloat16)
                        ad = jnp.abs(lb - pb)
                        a0, a1 = plsc.unpack(ad, format=plsc.PackFormat.INTERLEAVED)
                        accv = accv + a0 + a1
                    var = jnp.sum(accv)
                    accs = jnp.zeros((16,), jnp.float32)
                    for j in range(HW // 16, DW // 16):
                        l = ligbuf[e, pl.ds(j * 16, 16)]
                        p = protbuf[e, pl.ds(j * 16, 16)]
                        lb = plsc.bitcast(l, jnp.bfloat16)
                        pb = plsc.bitcast(p, jnp.bfloat16)
                        d0, d1 = plsc.unpack(lb - pb, format=plsc.PackFormat.INTERLEAVED)
                        accs = accs + d0 * d0
                        accs = accs + d1 * d1
                    ss = jnp.sum(accs)
                    msk = lanes == m
                    var_acc = jnp.where(msk, var, var_acc)
                    ss_acc = jnp.where(msk, ss, ss_acc)
                return var_acc, ss_acc

            z16 = jnp.zeros((16,), jnp.float32)
            var_acc, ss_acc = lax.fori_loop(0, 8, edge_pair, (z16, z16))

            i32 = plsc.bitcast(ss_acc, jnp.int32)
            i32 = jnp.int32(0x5F3759DF) - lax.shift_right_arithmetic(i32, 1)
            y = plsc.bitcast(i32, jnp.float32)
            for _ in range(3):
                y = y * (1.5 - 0.5 * ss_acc * y * y)
            mu_vec = jnp.where(ss_acc > 0.0, ss_acc * y, 0.0)

            mub[pl.ds(obase + g * 16, 16)] = mu_vec
            varb[pl.ds(obase + g * 16, 16)] = var_acc
            return carry

        lax.fori_loop(0, ngroups, group_body, 0)

    issue(0, B, ligA, protA, sla, spa)

    def pair_body(p, carry):
        k = 2 * p
        issue((k + 1) * B, B, ligB, protB, slb, spb)
        wait(k * B, B, ligA, protA, sla, spa)
        compute_block(k * B, ligA, protA, B // 16)

        @pl.when(k + 2 < NBW)
        def _():
            issue((k + 2) * B, B, ligA, protA, sla, spa)

        wait((k + 1) * B, B, ligB, protB, slb, spb)
        compute_block((k + 1) * B, ligB, protB, B // 16)
        return carry

    lax.fori_loop(0, NBW // 2, pair_body, 0)

    tl = ligA.at[pl.ds(0, TPAD), :]
    tp = protA.at[pl.ds(0, TPAD), :]
    issue(NBW * B, TPAD, tl, tp, sla, spa)
    wait(NBW * B, TPAD, tl, tp, sla, spa)
    compute_block(NBW * B, ligA, protA, 1)

    pltpu.sync_copy(mub.at[pl.ds(0, EPW)], mu_hbm.at[pl.ds(ebase, EPW)])
    pltpu.sync_copy(varb.at[pl.ds(0, EPW)], var_hbm.at[pl.ds(ebase, EPW)])


@functools.partial(
    pl.kernel,
    out_type=(
        jax.ShapeDtypeStruct((E,), jnp.float32),
        jax.ShapeDtypeStruct((E,), jnp.float32),
    ),
    mesh=plsc.VectorSubcoreMesh(core_axis_name="c", subcore_axis_name="s"),
    compiler_params=pltpu.CompilerParams(needs_layout_passes=False),
    scratch_types=[
        pltpu.VMEM((EBUF,), jnp.int32),
        pltpu.VMEM((EBUF,), jnp.int32),
        pltpu.VMEM((B, DW), jnp.float32),
        pltpu.VMEM((B, DW), jnp.float32),
        pltpu.VMEM((B, DW), jnp.float32),
        pltpu.VMEM((B, DW), jnp.float32),
        pltpu.VMEM((EBUF,), jnp.float32),
        pltpu.VMEM((EBUF,), jnp.float32),
        pltpu.SemaphoreType.DMA,
        pltpu.SemaphoreType.DMA,
        pltpu.SemaphoreType.DMA,
        pltpu.SemaphoreType.DMA,
    ],
)
def _sc_kernel(lig, prot, src, dst, mu_out, var_out,
               idxs, idxd, ligA, protA, ligB, protB, mub, varb,
               sla, spa, slb, spb):
    _body(lig, prot, src, dst, mu_out, var_out,
          idxs, idxd, ligA, protA, ligB, protB, mub, varb,
          sla, spa, slb, spb)


def _pack_words(x):
    xb = lax.bitcast_convert_type(x.astype(jnp.bfloat16), jnp.uint16)
    x32 = xb.astype(jnp.uint32)
    h = x.shape[1] // 4
    w_var = x32[:, 0:h] | (x32[:, h:2 * h] << 16)
    w_ss = x32[:, 2 * h:3 * h] | (x32[:, 3 * h:4 * h] << 16)
    return lax.bitcast_convert_type(
        jnp.concatenate([w_var, w_ss], axis=1), jnp.float32)


def kernel(ligand_h, protein_h, edge_index):
    ei = edge_index.astype(jnp.int32)
    mu, var = _sc_kernel(_pack_words(ligand_h), _pack_words(protein_h),
                         ei[0], ei[1])
    return jnp.stack([mu, var], axis=0).T
